# baseline (speedup 1.0000x reference)
import functools

import jax
import jax.numpy as jnp
from jax import lax
from jax.experimental import pallas as pl
from jax.experimental.pallas import tpu as pltpu
from jax.experimental.pallas import tpu_sc as plsc

CHUNK = 128


@functools.lru_cache(maxsize=None)
def _make_sc_kernel(V, B, NLAYERS, D):
    F = NLAYERS * D
    info = plsc.get_sparse_core_info()
    NC, NS, L = info.num_cores, info.num_subcores, info.num_lanes
    NW = NC * NS
    n_per = B // NW
    n_chunks = n_per // CHUNK
    groups = CHUNK // L
    NL = F // L

    mesh = plsc.VectorSubcoreMesh(core_axis_name="c", subcore_axis_name="s")

    @functools.partial(
        pl.kernel,
        out_type=jax.ShapeDtypeStruct((B,), jnp.float32),
        mesh=mesh,
        compiler_params=pltpu.CompilerParams(needs_layout_passes=False,
                                             use_tc_tiling_on_sc=False),
        scratch_types=[
            pltpu.VMEM((n_chunks, CHUNK), jnp.int32),
            pltpu.VMEM((n_chunks, CHUNK), jnp.int32),
            pltpu.VMEM((2, CHUNK, NLAYERS, D), jnp.float32),
            pltpu.VMEM((2, CHUNK, NLAYERS, D), jnp.float32),
            pltpu.VMEM((n_per,), jnp.float32),
            pltpu.VMEM((L * L,), jnp.float32),
            pltpu.VMEM((F,), jnp.float32),
            pltpu.VMEM((F,), jnp.float32),
            pltpu.SemaphoreType.DMA,
            pltpu.SemaphoreType.DMA,
        ],
    )
    def k(x_hbm, idsu_hbm, idsi_hbm, w1_hbm, w2_hbm, out_hbm,
          idx_u, idx_i, rows_u, rows_i, out_v, prod_v, wv1, wv2, sem0, sem1):
        wid = lax.axis_index("s") * NC + lax.axis_index("c")
        pltpu.sync_copy(idsu_hbm.at[pl.ds(wid * n_chunks, n_chunks)], idx_u)
        pltpu.sync_copy(idsi_hbm.at[pl.ds(wid * n_chunks, n_chunks)], idx_i)
        pltpu.sync_copy(w1_hbm, wv1)
        pltpu.sync_copy(w2_hbm, wv2)

        sems = (sem0, sem1)

        def fire(c):
            buf = c % 2
            du = pltpu.async_copy(x_hbm.at[idx_u.at[c]], rows_u.at[buf],
                                  sems[buf])
            di = pltpu.async_copy(x_hbm.at[idx_i.at[c]], rows_i.at[buf],
                                  sems[buf])
            return du, di

        w1l = [wv1[pl.ds(l * L, L)] for l in range(NL)]
        w2l = [wv2[pl.ds(l * L, L)] for l in range(NL)]

        def compute(c, buf):
            ru = rows_u.at[buf]
            ri = rows_i.at[buf]

            lane = lax.iota(jnp.int32, L)

            def body(g, carry):
                acc = jnp.zeros((L,), jnp.float32)
                for j in range(L):
                    e = g * L + j
                    uw = jnp.zeros((L,), jnp.float32)
                    iw = jnp.zeros((L,), jnp.float32)
                    for l in range(NL):
                        uw = uw + ru[e, l, pl.ds(0, L)] * w1l[l]
                        iw = iw + ri[e, l, pl.ds(0, L)] * w2l[l]
                    s = jnp.sum(uw * iw)
                    acc = jnp.where(lane == j, s, acc)
                out_v[pl.ds(c * CHUNK + g * L, L)] = acc
                return carry

            lax.fori_loop(0, groups, body, 0)

        descs = fire(0)
        for c in range(n_chunks):
            nxt = fire(c + 1) if c + 1 < n_chunks else None
            for d in descs:
                d.wait()
            compute(c, c % 2)
            descs = nxt

        pltpu.sync_copy(out_v, out_hbm.at[pl.ds(wid * n_per, n_per)])

    return k


def kernel(X, ids, w1, w2):
    V, NL, D = X.shape
    B = ids.shape[0]
    ids_u = ids[:, 0].reshape(-1, CHUNK)
    ids_i = ids[:, 1].reshape(-1, CHUNK)
    w1b = jnp.repeat(w1, D)
    w2b = jnp.repeat(w2, D)
    return _make_sc_kernel(V, B, NL, D)(X, ids_u, ids_i, w1b, w2b)


# speedup vs baseline: 5.1009x; 5.1009x over previous
import functools

import jax
import jax.numpy as jnp
from jax import lax
from jax.experimental import pallas as pl
from jax.experimental.pallas import tpu as pltpu
from jax.experimental.pallas import tpu_sc as plsc

STRIPE = 512
CHUNK = 128


@functools.lru_cache(maxsize=None)
def _detile_kernel(V, F):
    info = plsc.get_sparse_core_info()
    NC, NS, L = info.num_cores, info.num_subcores, info.num_lanes
    NW = NC * NS
    n_stripes = V // STRIPE
    vfull = n_stripes * STRIPE
    ntail = V - vfull
    k_iters = (n_stripes + NW - 1) // NW
    SZ = F * STRIPE

    mesh = plsc.VectorSubcoreMesh(core_axis_name="c", subcore_axis_name="s")

    @functools.partial(
        pl.kernel,
        out_type=jax.ShapeDtypeStruct((V * F,), jnp.float32),
        mesh=mesh,
        compiler_params=pltpu.CompilerParams(needs_layout_passes=False),
        scratch_types=[
            pltpu.VMEM((2, F, STRIPE), jnp.float32),
            pltpu.VMEM((2 * SZ,), jnp.float32),
            pltpu.VMEM((F * 64,), jnp.float32),
            pltpu.SemaphoreType.DMA,
            pltpu.SemaphoreType.DMA,
            pltpu.SemaphoreType.DMA,
            pltpu.SemaphoreType.DMA,
        ],
    )
    def k(x_hbm, xtail_hbm, out_hbm, stage, lin, tailbuf,
          semr, semw0, semw1, semt):
        wid = lax.axis_index("s") * NC + lax.axis_index("c")
        semw = (semw0, semw1)

        @pl.when(wid == 0)
        def _():
            pltpu.async_copy(xtail_hbm, tailbuf, semt).wait()

            def t_body(p, carry):
                pltpu.async_copy(
                    tailbuf.at[pl.ds(p * ntail, ntail)],
                    out_hbm.at[pl.ds(p * V + vfull, ntail)], semt)
                return carry

            lax.fori_loop(0, F, t_body, 0)

        def sb_of(k_idx):
            return wid + NW * k_idx

        def read(k_idx, buf, extra_cond=None):
            sb = sb_of(k_idx)
            cond = sb < n_stripes
            if extra_cond is not None:
                cond = jnp.logical_and(cond, extra_cond)

            @pl.when(cond)
            def _():
                pltpu.async_copy(
                    x_hbm.at[:, pl.ds(pl.multiple_of(sb * STRIPE, 128),
                                      STRIPE)],
                    stage.at[buf], semr)

        def drain_read(k_idx, buf):
            sb = sb_of(k_idx)

            @pl.when(sb < n_stripes)
            def _():
                pltpu.make_async_copy(x_hbm.at[:, pl.ds(0, STRIPE)],
                                      stage.at[buf], semr).wait()

        def detile(k_idx, buf):
            sb = sb_of(k_idx)

            @pl.when(sb < n_stripes)
            def _():
                def c_body(p, carry):
                    for j in range(STRIPE // L):
                        lin[pl.ds(buf * SZ + p * STRIPE + j * L, L)] = (
                            stage[buf, p, pl.ds(j * L, L)])
                    return carry

                lax.fori_loop(0, F, c_body, 0)

        def write(k_idx, buf):
            sb = sb_of(k_idx)

            @pl.when(sb < n_stripes)
            def _():
                def w_body(p, carry):
                    pltpu.async_copy(
                        lin.at[pl.ds(buf * SZ + p * STRIPE, STRIPE)],
                        out_hbm.at[pl.ds(p * V + sb * STRIPE, STRIPE)],
                        semw[buf])
                    return carry

                lax.fori_loop(0, F, w_body, 0)

        def drain_write(k_idx, buf, extra_cond=None):
            sb = sb_of(k_idx)
            cond = sb < n_stripes
            if extra_cond is not None:
                cond = jnp.logical_and(cond, extra_cond)

            @pl.when(cond)
            def _():
                pltpu.make_async_copy(out_hbm.at[pl.ds(0, SZ)],
                                      lin.at[pl.ds(buf * SZ, SZ)],
                                      semw[buf]).wait()

        read(0, 0)
        read(1, 1)

        def kk_body(kk, carry):
            for off in (0, 1):
                ki = 2 * kk + off
                buf = off
                drain_read(ki, buf)
                drain_write(ki - 2, buf, extra_cond=(ki >= 2))
                detile(ki, buf)
                write(ki, buf)
                read(ki + 2, buf)
            return carry

        lax.fori_loop(0, k_iters // 2, kk_body, 0)
        if k_iters >= 2:
            drain_write(k_iters - 2, k_iters % 2)
        drain_write(k_iters - 1, (k_iters - 1) % 2)

        @pl.when(wid == 0)
        def _():
            pltpu.make_async_copy(xtail_hbm, tailbuf, semt).wait()

    return k


@functools.lru_cache(maxsize=None)
def _gather_kernel(V, B, NLAYERS, D):
    F = NLAYERS * D
    info = plsc.get_sparse_core_info()
    NC, NS, L = info.num_cores, info.num_subcores, info.num_lanes
    NW = NC * NS
    n_per = B // NW
    n_chunks = n_per // CHUNK
    groups = CHUNK // L
    SZ = F * CHUNK

    mesh = plsc.VectorSubcoreMesh(core_axis_name="c", subcore_axis_name="s")

    @functools.partial(
        pl.kernel,
        out_type=jax.ShapeDtypeStruct((B,), jnp.float32),
        mesh=mesh,
        compiler_params=pltpu.CompilerParams(needs_layout_passes=False),
        scratch_types=[
            pltpu.VMEM((n_per,), jnp.int32),
            pltpu.VMEM((n_per,), jnp.int32),
            pltpu.VMEM((2 * SZ,), jnp.float32),
            pltpu.VMEM((2 * SZ,), jnp.float32),
            pltpu.VMEM((n_per,), jnp.float32),
            pltpu.VMEM((F,), jnp.float32),
            pltpu.VMEM((F,), jnp.float32),
            pltpu.SemaphoreType.DMA,
            pltpu.SemaphoreType.DMA,
        ],
    )
    def k(t_hbm, idsu_hbm, idsi_hbm, w1_hbm, w2_hbm, out_hbm,
          idx_u, idx_i, cols_u, cols_i, out_v, wv1, wv2, sem0, sem1):
        wid = lax.axis_index("s") * NC + lax.axis_index("c")
        pltpu.sync_copy(idsu_hbm.at[pl.ds(wid * n_per, n_per)], idx_u)
        pltpu.sync_copy(idsi_hbm.at[pl.ds(wid * n_per, n_per)], idx_i)
        pltpu.sync_copy(w1_hbm, wv1)
        pltpu.sync_copy(w2_hbm, wv2)
        sems = (sem0, sem1)

        def issue(c):
            buf = c % 2
            sem = sems[buf]
            iu = idx_u.at[pl.ds(c * CHUNK, CHUNK)]
            ii = idx_i.at[pl.ds(c * CHUNK, CHUNK)]

            def p_body(p, carry):
                tbl = t_hbm.at[pl.ds(pl.multiple_of(p * V, 8), V)]
                pltpu.async_copy(
                    tbl.at[iu],
                    cols_u.at[pl.ds(buf * SZ + p * CHUNK, CHUNK)], sem)
                pltpu.async_copy(
                    tbl.at[ii],
                    cols_i.at[pl.ds(buf * SZ + p * CHUNK, CHUNK)], sem)
                return carry

            lax.fori_loop(0, F, p_body, 0)

        def drain(c):
            buf = c % 2
            src = t_hbm.at[pl.ds(0, SZ)]
            pltpu.make_async_copy(src, cols_u.at[pl.ds(buf * SZ, SZ)],
                                  sems[buf]).wait()
            pltpu.make_async_copy(src, cols_i.at[pl.ds(buf * SZ, SZ)],
                                  sems[buf]).wait()

        w1l = [wv1[pl.ds(l * L, L)] for l in range(NLAYERS)]
        w2l = [wv2[pl.ds(l * L, L)] for l in range(NLAYERS)]

        def compute(c, buf):
            def body(g, carry):
                acc = jnp.zeros((L,), jnp.float32)
                for d in range(D):
                    uw = jnp.zeros((L,), jnp.float32)
                    iw = jnp.zeros((L,), jnp.float32)
                    for l in range(NLAYERS):
                        o = buf * SZ + (l * D + d) * CHUNK + g * L
                        uw = uw + cols_u[pl.ds(o, L)] * w1l[l]
                        iw = iw + cols_i[pl.ds(o, L)] * w2l[l]
                    acc = acc + uw * iw
                out_v[pl.ds(c * CHUNK + g * L, L)] = acc
                return carry

            lax.fori_loop(0, groups, body, 0)

        issue(0)
        for c in range(n_chunks):
            drain(c)
            if c + 1 < n_chunks:
                issue(c + 1)
            compute(c, c % 2)

        pltpu.sync_copy(out_v, out_hbm.at[pl.ds(wid * n_per, n_per)])

    return k


def kernel(X, ids, w1, w2):
    V, NL, D = X.shape
    B = ids.shape[0]
    F = NL * D
    x2 = jnp.transpose(X, (1, 2, 0)).reshape(F, V)
    vfull = (V // STRIPE) * STRIPE
    xtail = jnp.transpose(X[vfull:], (1, 2, 0)).reshape(-1)
    t1d = _detile_kernel(V, F)(x2, xtail)
    ids_u = ids[:, 0]
    ids_i = ids[:, 1]
    w1b = jnp.repeat(w1, D)
    w2b = jnp.repeat(w2, D)
    return _gather_kernel(V, B, NL, D)(t1d, ids_u, ids_i, w1b, w2b)


# speedup vs baseline: 10.3634x; 2.0317x over previous
import functools

import jax
import jax.numpy as jnp
from jax import lax
from jax.experimental import pallas as pl
from jax.experimental.pallas import tpu as pltpu
from jax.experimental.pallas import tpu_sc as plsc

STRIPE = 512
LCAP = 1536
TCAP = 64


@functools.lru_cache(maxsize=None)
def _sweep_kernel(V, B, F):
    info = plsc.get_sparse_core_info()
    NC, NS, L = info.num_cores, info.num_subcores, info.num_lanes
    NW = NC * NS
    n_stripes = V // STRIPE
    vfull = n_stripes * STRIPE
    ntail = V - vfull
    k_iters = (n_stripes + NW - 1) // NW
    if k_iters % 2:
        k_iters += 1
    nidv = 2 * B // L
    SHIFT_OWN = 9
    SHIFT_K = 14

    mesh = plsc.VectorSubcoreMesh(core_axis_name="c", subcore_axis_name="s")

    @functools.partial(
        pl.kernel,
        out_type=jax.ShapeDtypeStruct((2 * B * F,), jnp.float32),
        mesh=mesh,
        compiler_params=pltpu.CompilerParams(needs_layout_passes=False),
        scratch_types=[
            pltpu.VMEM((2, F, STRIPE), jnp.float32),
            pltpu.VMEM((2 * B,), jnp.int32),
            pltpu.VMEM((LCAP,), jnp.int32),
            pltpu.VMEM((LCAP,), jnp.int32),
            pltpu.VMEM((TCAP,), jnp.int32),
            pltpu.VMEM((TCAP,), jnp.int32),
            pltpu.VMEM((TCAP * F,), jnp.float32),
            pltpu.VMEM((F * 64,), jnp.float32),
            pltpu.SemaphoreType.DMA,
            pltpu.SemaphoreType.DMA,
        ],
    )
    def k(x_hbm, xtail_hbm, idsu_hbm, idsi_hbm, out_hbm,
          stage, idsv, lid, lslot, tid, tslot, rowbuf, tailvm, semr, semo):
        wid = lax.axis_index("s") * NC + lax.axis_index("c")

        def read(k_idx, buf):
            sb = wid + NW * k_idx

            @pl.when(sb < n_stripes)
            def _():
                pltpu.async_copy(
                    x_hbm.at[:, pl.ds(pl.multiple_of(sb * STRIPE, 128),
                                      STRIPE)],
                    stage.at[buf], semr)

        def drain_read(k_idx, buf):
            sb = wid + NW * k_idx

            @pl.when(sb < n_stripes)
            def _():
                pltpu.make_async_copy(x_hbm.at[:, pl.ds(0, STRIPE)],
                                      stage.at[buf], semr).wait()

        read(0, 0)
        read(1, 1)

        pltpu.sync_copy(idsu_hbm, idsv.at[pl.ds(0, B)])
        pltpu.sync_copy(idsi_hbm, idsv.at[pl.ds(B, B)])

        @pl.when(wid == jnp.int32(n_stripes % NW))
        def _():
            pltpu.sync_copy(xtail_hbm, tailvm)

        def s_body(i, carry):
            lid[pl.ds(i * L, L)] = jnp.full((L,), jnp.int32(0x7FFFFFFF))
            return carry

        lax.fori_loop(0, LCAP // L, s_body, 0)

        iota = lax.iota(jnp.int32, L)

        def p_body(i, off):
            vec = idsv[pl.ds(i * L, L)]
            m = ((vec >> SHIFT_OWN) & jnp.int32(NW - 1)) == wid
            pos = off + plsc.cumsum(m.astype(jnp.int32)) - 1
            m = jnp.logical_and(m, pos < LCAP)
            plsc.store_scatter(lid, [pos], vec, mask=m)
            plsc.store_scatter(lslot, [pos], i * L + iota, mask=m)
            return off + plsc.all_reduce_population_count(m)[0]

        nown = lax.fori_loop(0, nidv, p_body, jnp.int32(0))
        nscan = (nown + (L - 1)) // L

        def serve(k_target, gather_fn, prev_cnt):
            def c_body(i, cnt):
                vec = lid[pl.ds(i * L, L)]
                m = (vec >> SHIFT_K) == k_target
                pos = cnt + plsc.cumsum(m.astype(jnp.int32)) - 1
                plsc.store_scatter(tid, [pos], vec, mask=m)
                plsc.store_scatter(tslot, [pos],
                                   lslot[pl.ds(i * L, L)], mask=m)
                return cnt + plsc.all_reduce_population_count(m)[0]

            cnt = lax.fori_loop(0, nscan, c_body, jnp.int32(0))

            def d_body(i, carry):
                pltpu.make_async_copy(out_hbm.at[pl.ds(0, F)],
                                      rowbuf.at[pl.ds(0, F)], semo).wait()
                return carry

            lax.fori_loop(0, prev_cnt, d_body, 0)

            def b_body(b, carry):
                idv = tid[pl.ds(b * L, L)]
                slv = tslot[pl.ds(b * L, L)]
                for j in range(L):
                    e = b * L + j

                    @pl.when(e < cnt)
                    def _():
                        one_id = idv[j]
                        sg = slv[j]
                        roff = e * F
                        gather_fn(one_id, roff)
                        pltpu.async_copy(
                            rowbuf.at[pl.ds(roff, F)],
                            out_hbm.at[pl.ds(sg * F, F)], semo)
                return carry

            lax.fori_loop(0, (cnt + (L - 1)) // L, b_body, 0)
            return cnt

        def make_stage_gather(buf):
            def g(one_id, roff):
                col = jnp.full((L,), 0, jnp.int32) + (
                    one_id & jnp.int32(STRIPE - 1))
                for l in range(F // L):
                    pv = l * L + iota
                    rowbuf[pl.ds(roff + l * L, L)] = plsc.load_gather(
                        stage.at[buf], [pv, col])
            return g

        def kk_body(kk, prev_cnt):
            for off in (0, 1):
                ki = 2 * kk + off
                buf = off
                drain_read(ki, buf)
                prev_cnt = serve(ki, make_stage_gather(buf), prev_cnt)
                read(ki + 2, buf)
            return prev_cnt

        prev_cnt = lax.fori_loop(0, k_iters // 2, kk_body, jnp.int32(0))

        tail_owner = n_stripes % NW
        tail_k = n_stripes // NW

        def tail_gather(one_id, roff):
            c = one_id - jnp.int32(vfull)
            for l in range(F // L):
                pv = (l * L + iota) * ntail + c
                rowbuf[pl.ds(roff + l * L, L)] = plsc.load_gather(
                    tailvm, [pv])

        def final_drain(n, carry_unused=None):
            def d_body(i, carry):
                pltpu.make_async_copy(out_hbm.at[pl.ds(0, F)],
                                      rowbuf.at[pl.ds(0, F)], semo).wait()
                return carry

            lax.fori_loop(0, n, d_body, 0)

        @pl.when(wid == jnp.int32(tail_owner))
        def _():
            cnt = serve(jnp.int32(tail_k), tail_gather, prev_cnt)
            final_drain(cnt)

        @pl.when(wid != jnp.int32(tail_owner))
        def _():
            final_drain(prev_cnt)

    return k


@functools.lru_cache(maxsize=None)
def _dot_kernel(B, NLAYERS, D):
    F = NLAYERS * D
    info = plsc.get_sparse_core_info()
    NC, NS, L = info.num_cores, info.num_subcores, info.num_lanes
    NW = NC * NS
    n_per = B // NW

    mesh = plsc.VectorSubcoreMesh(core_axis_name="c", subcore_axis_name="s")

    @functools.partial(
        pl.kernel,
        out_type=jax.ShapeDtypeStruct((B,), jnp.float32),
        mesh=mesh,
        compiler_params=pltpu.CompilerParams(needs_layout_passes=False),
        scratch_types=[
            pltpu.VMEM((n_per * F,), jnp.float32),
            pltpu.VMEM((n_per * F,), jnp.float32),
            pltpu.VMEM((n_per,), jnp.float32),
            pltpu.VMEM((F,), jnp.float32),
            pltpu.VMEM((F,), jnp.float32),
        ],
    )
    def k(rows_hbm, w1_hbm, w2_hbm, out_hbm, ru, ri, out_v, wv1, wv2):
        wid = lax.axis_index("s") * NC + lax.axis_index("c")
        pltpu.sync_copy(rows_hbm.at[pl.ds(wid * n_per * F, n_per * F)], ru)
        pltpu.sync_copy(
            rows_hbm.at[pl.ds((B + wid * n_per) * F, n_per * F)], ri)
        pltpu.sync_copy(w1_hbm, wv1)
        pltpu.sync_copy(w2_hbm, wv2)

        w1l = [wv1[pl.ds(l * L, L)] for l in range(NLAYERS)]
        w2l = [wv2[pl.ds(l * L, L)] for l in range(NLAYERS)]
        lane = lax.iota(jnp.int32, L)

        def body(g, carry):
            acc = jnp.zeros((L,), jnp.float32)
            for j in range(L):
                e = (g * L + j) * F
                uw = jnp.zeros((L,), jnp.float32)
                iw = jnp.zeros((L,), jnp.float32)
                for l in range(NLAYERS):
                    uw = uw + ru[pl.ds(e + l * L, L)] * w1l[l]
                    iw = iw + ri[pl.ds(e + l * L, L)] * w2l[l]
                s = jnp.sum(uw * iw)
                acc = jnp.where(lane == j, s, acc)
            out_v[pl.ds(g * L, L)] = acc
            return carry

        lax.fori_loop(0, n_per // L, body, 0)
        pltpu.sync_copy(out_v, out_hbm.at[pl.ds(wid * n_per, n_per)])

    return k


def kernel(X, ids, w1, w2):
    V, NL, D = X.shape
    B = ids.shape[0]
    F = NL * D
    x2 = jnp.transpose(X, (1, 2, 0)).reshape(F, V)
    vfull = (V // STRIPE) * STRIPE
    xtail = jnp.transpose(X[vfull:], (1, 2, 0)).reshape(-1)
    ids_u = ids[:, 0]
    ids_i = ids[:, 1]
    rows = _sweep_kernel(V, B, F)(x2, xtail, ids_u, ids_i)
    w1b = jnp.repeat(w1, D)
    w2b = jnp.repeat(w2, D)
    return _dot_kernel(B, NL, D)(rows, w1b, w2b)
